# baseline (speedup 1.0000x reference)
import jax
import jax.numpy as jnp
from jax import lax
from jax.experimental import pallas as pl
from jax.experimental.pallas import tpu as pltpu
from jax.experimental.pallas import tpu_sc as plsc

NC = 2
NS = 16
NL = 16
PAD = 32
CHUNK = 128
IDXG = 8
K = 4


def _dense_pre(x, W_l):
    n, d = x.shape
    d2 = d + PAD
    blk = 1000
    grid = n // blk

    def body(x_ref, wl_ref, h2_ref):
        xb = x_ref[...]
        dn = (((1,), (1,)), ((), ()))
        h = lax.dot_general(xb, wl_ref[...], dn)
        lane = lax.broadcasted_iota(jnp.int32, (blk, PAD), 1)
        tail = jnp.where(lane == 0, 1.0, 0.0).astype(jnp.float32)
        h2_ref[...] = jnp.concatenate([h, tail], axis=1).astype(jnp.bfloat16)

    return pl.pallas_call(
        body,
        grid=(grid,),
        in_specs=[
            pl.BlockSpec((blk, d), lambda i: (i, 0)),
            pl.BlockSpec((d, d), lambda i: (0, 0)),
        ],
        out_specs=pl.BlockSpec((blk, d2), lambda i: (i, 0)),
        out_shape=jax.ShapeDtypeStruct((n, d2), jnp.bfloat16),
    )(x, W_l)


def _sc_aggregate(h2, src2d, dst2d, npt, nr):
    d2 = h2.shape[1]
    nb = npt // IDXG
    assert nb >= 4 and nb % 2 == 0 and IDXG % K == 0

    def body(h2_hbm, src_hbm, dst_hbm, zagg_hbm, agg_out,
             srcb0, dstb0, srcb1, dstb1, rows0, rows1, rows2, rows3, agg_sh,
             gsem0, gsem1, gsem2, gsem3, ssem0, ssem1, ssem2, ssem3,
             isem0, isem1):
        cid = lax.axis_index("c")
        sid = lax.axis_index("s")
        wid = cid * NS + sid
        row0 = wid * npt

        rpt = nr // NS
        r0 = sid * rpt
        pltpu.sync_copy(zagg_hbm.at[pl.ds(r0, rpt)],
                        agg_sh.at[pl.ds(r0, rpt)])
        plsc.subcore_barrier()

        srcb = (srcb0, srcb1)
        dstb = (dstb0, dstb1)
        rows = (rows0, rows1, rows2, rows3)
        gsem = (gsem0, gsem1, gsem2, gsem3)
        ssem = (ssem0, ssem1, ssem2, ssem3)
        isem = (isem0, isem1)

        def idx_load(b, p, sync=False):
            if sync:
                pltpu.sync_copy(src_hbm.at[pl.ds(row0 + b * IDXG, IDXG)], srcb[p])
                pltpu.sync_copy(dst_hbm.at[pl.ds(row0 + b * IDXG, IDXG)], dstb[p])
            else:
                pltpu.async_copy(src_hbm.at[pl.ds(row0 + b * IDXG, IDXG)],
                                 srcb[p], isem[p])
                pltpu.async_copy(dst_hbm.at[pl.ds(row0 + b * IDXG, IDXG)],
                                 dstb[p], isem[p])

        def idx_wait(p):
            pltpu.make_async_copy(src_hbm.at[pl.ds(row0, IDXG)],
                                  srcb[p], isem[p]).wait()
            pltpu.make_async_copy(dst_hbm.at[pl.ds(row0, IDXG)],
                                  dstb[p], isem[p]).wait()

        def gather(k, p, g):
            pltpu.async_copy(h2_hbm.at[srcb[p].at[g]], rows[k], gsem[k])

        def gather_wait(k):
            pltpu.make_async_copy(h2_hbm.at[pl.ds(0, CHUNK)],
                                  rows[k], gsem[k]).wait()

        def scatter(k, p, g):
            pltpu.async_copy(rows[k], agg_sh.at[dstb[p].at[g]], ssem[k],
                             add=True)

        def scatter_wait(k):
            pltpu.make_async_copy(h2_hbm.at[pl.ds(0, CHUNK)],
                                  rows[k], ssem[k]).wait()

        def block_step(b, p, g, first_block):
            k = g % K
            kprev = (g - 1) % K
            if not (first_block and g == 0):
                scatter_wait(kprev)
            if g == 0:
                idx_load(b + 1, 1 - p)
            if g + K - 1 < IDXG:
                gather(kprev, p, g + K - 1)
            else:
                if g == IDXG - K + 1:
                    idx_wait(1 - p)
                gather(kprev, 1 - p, g + K - 1 - IDXG)
            gather_wait(k)
            scatter(k, p, g)

        idx_load(0, 0, sync=True)
        for k in range(K - 1):
            gather(k, 0, k)
        for g in range(IDXG):
            block_step(0, 0, g, True)
        for g in range(IDXG):
            block_step(1, 1, g, False)

        @pl.loop(0, (nb - 2) // 2)
        def _(b2):
            b = 2 + 2 * b2
            for g in range(IDXG):
                block_step(b, 0, g, False)
            for g in range(IDXG):
                block_step(b + 1, 1, g, False)

        for k in range(K - 1):
            gather_wait(k)
        scatter_wait(K - 1)
        plsc.subcore_barrier()
        pltpu.sync_copy(agg_sh.at[pl.ds(r0, rpt)],
                        agg_out.at[cid, pl.ds(r0, rpt)])

    mesh = plsc.VectorSubcoreMesh(core_axis_name="c", subcore_axis_name="s",
                                  num_cores=NC, num_subcores=NS)
    zagg = jnp.zeros((nr, d2), jnp.bfloat16)
    dma = pltpu.SemaphoreType.DMA
    return pl.kernel(
        body,
        out_type=jax.ShapeDtypeStruct((NC, nr, d2), jnp.bfloat16),
        mesh=mesh,
        compiler_params=pltpu.CompilerParams(use_tc_tiling_on_sc=False),
        scratch_types=[
            pltpu.VMEM((IDXG, CHUNK), jnp.int32),
            pltpu.VMEM((IDXG, CHUNK), jnp.int32),
            pltpu.VMEM((IDXG, CHUNK), jnp.int32),
            pltpu.VMEM((IDXG, CHUNK), jnp.int32),
            pltpu.VMEM((CHUNK, d2), jnp.bfloat16),
            pltpu.VMEM((CHUNK, d2), jnp.bfloat16),
            pltpu.VMEM((CHUNK, d2), jnp.bfloat16),
            pltpu.VMEM((CHUNK, d2), jnp.bfloat16),
            pltpu.VMEM_SHARED((nr, d2), jnp.bfloat16),
            dma, dma, dma, dma, dma, dma, dma, dma, dma, dma,
        ],
    )(h2, src2d, dst2d, zagg)


def _post(agg, x, W_r, b_l):
    n, d = x.shape
    blk = 1000
    grid = n // blk
    d2 = d + PAD

    def body(a0, a1, x_ref, wr_ref, b_ref, o_ref):
        cnt = (a0[0, :, d:d + 1] + a1[0, :, d:d + 1]).astype(jnp.float32)
        denom = jnp.maximum(cnt, 1.0)
        s = a0[0, :, :d].astype(jnp.float32) + a1[0, :, :d].astype(jnp.float32)
        dn = (((1,), (1,)), ((), ()))
        r = lax.dot_general(x_ref[...], wr_ref[...], dn) + b_ref[...]
        o_ref[...] = s / denom + r

    return pl.pallas_call(
        body,
        grid=(grid,),
        in_specs=[
            pl.BlockSpec((1, blk, d2), lambda i: (0, i, 0)),
            pl.BlockSpec((1, blk, d2), lambda i: (1, i, 0)),
            pl.BlockSpec((blk, d), lambda i: (i, 0)),
            pl.BlockSpec((d, d), lambda i: (0, 0)),
            pl.BlockSpec((1, d), lambda i: (0, 0)),
        ],
        out_specs=pl.BlockSpec((blk, d), lambda i: (i, 0)),
        out_shape=jax.ShapeDtypeStruct((n, d), jnp.float32),
    )(agg, agg, x, W_r, b_l.reshape(1, d))


def kernel(x, edge_index, W_l, b_l, W_r):
    n, d = x.shape
    e = edge_index.shape[1]
    src = edge_index[0].astype(jnp.int32)
    dst = edge_index[1].astype(jnp.int32)

    nw = NC * NS
    npt = -(-e // (nw * CHUNK * 2 * IDXG)) * 2 * IDXG
    ep = nw * npt * CHUNK
    nr = -(-(n + 1) // (NS * 8)) * NS * 8

    pad = ep + IDXG * CHUNK - e
    src_p = jnp.concatenate([src, jnp.zeros((pad,), jnp.int32)])
    dst_p = jnp.concatenate([dst, jnp.full((pad,), n, jnp.int32)])
    src2d = src_p.reshape(nw * npt + IDXG, CHUNK)
    dst2d = dst_p.reshape(nw * npt + IDXG, CHUNK)

    h2 = _dense_pre(x, W_l)
    agg = _sc_aggregate(h2, src2d, dst2d, npt, nr)
    out = _post(agg, x, W_r, b_l)
    return out


# speedup vs baseline: 1.1136x; 1.1136x over previous
import jax
import jax.numpy as jnp
from jax import lax
from jax.experimental import pallas as pl
from jax.experimental.pallas import tpu as pltpu
from jax.experimental.pallas import tpu_sc as plsc

NC = 2
NS = 16
NL = 16
PAD = 32
CHUNK = 128
IDXG = 8
K = 4


def _dense_pre(x, W_l, b_l, W_r):
    n, d = x.shape
    d2 = d + PAD
    blk = 1000
    grid = n // blk

    def body(x_ref, wl_ref, wr_ref, b_ref, h2_ref, r_ref):
        xb = x_ref[...]
        dn = (((1,), (1,)), ((), ()))
        h = lax.dot_general(xb, wl_ref[...], dn)
        lane = lax.broadcasted_iota(jnp.int32, (blk, PAD), 1)
        tail = jnp.where(lane == 0, 1.0, 0.0).astype(jnp.float32)
        h2_ref[...] = jnp.concatenate([h, tail], axis=1).astype(jnp.bfloat16)
        r_ref[...] = lax.dot_general(xb, wr_ref[...], dn) + b_ref[...]

    h2, r = pl.pallas_call(
        body,
        grid=(grid,),
        in_specs=[
            pl.BlockSpec((blk, d), lambda i: (i, 0)),
            pl.BlockSpec((d, d), lambda i: (0, 0)),
            pl.BlockSpec((d, d), lambda i: (0, 0)),
            pl.BlockSpec((1, d), lambda i: (0, 0)),
        ],
        out_specs=[
            pl.BlockSpec((blk, d2), lambda i: (i, 0)),
            pl.BlockSpec((blk, d), lambda i: (i, 0)),
        ],
        out_shape=[
            jax.ShapeDtypeStruct((n, d2), jnp.bfloat16),
            jax.ShapeDtypeStruct((n, d), jnp.float32),
        ],
    )(x, W_l, W_r, b_l.reshape(1, d))
    return h2, r


def _sc_aggregate(h2, src2d, dst2d, npt, nr):
    d2 = h2.shape[1]
    nb = npt // IDXG
    assert nb >= 4 and nb % 2 == 0 and IDXG % K == 0

    def body(h2_hbm, src_hbm, dst_hbm, zagg_hbm, agg_out,
             srcb0, dstb0, srcb1, dstb1, rows0, rows1, rows2, rows3, agg_sh,
             gsem0, gsem1, gsem2, gsem3, ssem0, ssem1, ssem2, ssem3,
             isem0, isem1):
        cid = lax.axis_index("c")
        sid = lax.axis_index("s")
        wid = cid * NS + sid
        row0 = wid * npt

        rpt = nr // NS
        r0 = sid * rpt
        pltpu.sync_copy(zagg_hbm.at[pl.ds(r0, rpt)],
                        agg_sh.at[pl.ds(r0, rpt)])
        plsc.subcore_barrier()

        srcb = (srcb0, srcb1)
        dstb = (dstb0, dstb1)
        rows = (rows0, rows1, rows2, rows3)
        gsem = (gsem0, gsem1, gsem2, gsem3)
        ssem = (ssem0, ssem1, ssem2, ssem3)
        isem = (isem0, isem1)

        def idx_load(b, p, sync=False):
            if sync:
                pltpu.sync_copy(src_hbm.at[pl.ds(row0 + b * IDXG, IDXG)], srcb[p])
                pltpu.sync_copy(dst_hbm.at[pl.ds(row0 + b * IDXG, IDXG)], dstb[p])
            else:
                pltpu.async_copy(src_hbm.at[pl.ds(row0 + b * IDXG, IDXG)],
                                 srcb[p], isem[p])
                pltpu.async_copy(dst_hbm.at[pl.ds(row0 + b * IDXG, IDXG)],
                                 dstb[p], isem[p])

        def idx_wait(p):
            pltpu.make_async_copy(src_hbm.at[pl.ds(row0, IDXG)],
                                  srcb[p], isem[p]).wait()
            pltpu.make_async_copy(dst_hbm.at[pl.ds(row0, IDXG)],
                                  dstb[p], isem[p]).wait()

        def gather(k, p, g):
            pltpu.async_copy(h2_hbm.at[srcb[p].at[g]], rows[k], gsem[k])

        def gather_wait(k):
            pltpu.make_async_copy(h2_hbm.at[pl.ds(0, CHUNK)],
                                  rows[k], gsem[k]).wait()

        def scatter(k, p, g):
            pltpu.async_copy(rows[k], agg_sh.at[dstb[p].at[g]], ssem[k],
                             add=True)

        def scatter_wait(k):
            pltpu.make_async_copy(h2_hbm.at[pl.ds(0, CHUNK)],
                                  rows[k], ssem[k]).wait()

        def block_step(b, p, g, first_block):
            k = g % K
            kprev = (g - 1) % K
            if not (first_block and g == 0):
                scatter_wait(kprev)
            if g == 0:
                idx_load(b + 1, 1 - p)
            if g + K - 1 < IDXG:
                gather(kprev, p, g + K - 1)
            else:
                if g == IDXG - K + 1:
                    idx_wait(1 - p)
                gather(kprev, 1 - p, g + K - 1 - IDXG)
            gather_wait(k)
            scatter(k, p, g)

        idx_load(0, 0, sync=True)
        for k in range(K - 1):
            gather(k, 0, k)
        for g in range(IDXG):
            block_step(0, 0, g, True)
        for g in range(IDXG):
            block_step(1, 1, g, False)

        @pl.loop(0, (nb - 2) // 2)
        def _(b2):
            b = 2 + 2 * b2
            for g in range(IDXG):
                block_step(b, 0, g, False)
            for g in range(IDXG):
                block_step(b + 1, 1, g, False)

        for k in range(K - 1):
            gather_wait(k)
        scatter_wait(K - 1)
        plsc.subcore_barrier()
        pltpu.sync_copy(agg_sh.at[pl.ds(r0, rpt)],
                        agg_out.at[cid, pl.ds(r0, rpt)])

    mesh = plsc.VectorSubcoreMesh(core_axis_name="c", subcore_axis_name="s",
                                  num_cores=NC, num_subcores=NS)
    zagg = jnp.zeros((nr, d2), jnp.bfloat16)
    dma = pltpu.SemaphoreType.DMA
    return pl.kernel(
        body,
        out_type=jax.ShapeDtypeStruct((NC, nr, d2), jnp.bfloat16),
        mesh=mesh,
        compiler_params=pltpu.CompilerParams(use_tc_tiling_on_sc=False),
        scratch_types=[
            pltpu.VMEM((IDXG, CHUNK), jnp.int32),
            pltpu.VMEM((IDXG, CHUNK), jnp.int32),
            pltpu.VMEM((IDXG, CHUNK), jnp.int32),
            pltpu.VMEM((IDXG, CHUNK), jnp.int32),
            pltpu.VMEM((CHUNK, d2), jnp.bfloat16),
            pltpu.VMEM((CHUNK, d2), jnp.bfloat16),
            pltpu.VMEM((CHUNK, d2), jnp.bfloat16),
            pltpu.VMEM((CHUNK, d2), jnp.bfloat16),
            pltpu.VMEM_SHARED((nr, d2), jnp.bfloat16),
            dma, dma, dma, dma, dma, dma, dma, dma, dma, dma,
        ],
    )(h2, src2d, dst2d, zagg)


def _post(agg, r):
    n, d = r.shape
    blk = 1000
    grid = n // blk

    d2 = d + PAD

    def body(a0, a1, r_ref, o_ref):
        cnt = (a0[0, :, d:d + 1] + a1[0, :, d:d + 1]).astype(jnp.float32)
        denom = jnp.maximum(cnt, 1.0)
        s = a0[0, :, :d].astype(jnp.float32) + a1[0, :, :d].astype(jnp.float32)
        o_ref[...] = s / denom + r_ref[...]

    return pl.pallas_call(
        body,
        grid=(grid,),
        in_specs=[
            pl.BlockSpec((1, blk, d2), lambda i: (0, i, 0)),
            pl.BlockSpec((1, blk, d2), lambda i: (1, i, 0)),
            pl.BlockSpec((blk, d), lambda i: (i, 0)),
        ],
        out_specs=pl.BlockSpec((blk, d), lambda i: (i, 0)),
        out_shape=jax.ShapeDtypeStruct((n, d), jnp.float32),
    )(agg, agg, r)


def kernel(x, edge_index, W_l, b_l, W_r):
    n, d = x.shape
    e = edge_index.shape[1]
    src = edge_index[0].astype(jnp.int32)
    dst = edge_index[1].astype(jnp.int32)

    nw = NC * NS
    npt = -(-e // (nw * CHUNK * 2 * IDXG)) * 2 * IDXG
    ep = nw * npt * CHUNK
    nr = -(-(n + 1) // (NS * 8)) * NS * 8

    pad = ep + IDXG * CHUNK - e
    src_p = jnp.concatenate([src, jnp.zeros((pad,), jnp.int32)])
    dst_p = jnp.concatenate([dst, jnp.full((pad,), n, jnp.int32)])
    src2d = src_p.reshape(nw * npt + IDXG, CHUNK)
    dst2d = dst_p.reshape(nw * npt + IDXG, CHUNK)

    h2, r = _dense_pre(x, W_l, b_l, W_r)
    agg = _sc_aggregate(h2, src2d, dst2d, npt, nr)
    out = _post(agg, r)
    return out
